# baseline (speedup 1.0000x reference)
import functools

import jax
import jax.numpy as jnp
from jax import lax
from jax.experimental import pallas as pl
from jax.experimental.pallas import tpu as pltpu
from jax.experimental.pallas import tpu_sc as plsc

B = 16384
F = 32
NC = 2
NS = 16
NW = NC * NS
BPW = B // NW
CHUNK = 128
NCHUNK = BPW // CHUNK
LANES = 16
NBLK = BPW // LANES

_mesh = plsc.VectorSubcoreMesh(core_axis_name="c", subcore_axis_name="s")


@functools.partial(
    pl.kernel,
    mesh=_mesh,
    compiler_params=pltpu.CompilerParams(use_tc_tiling_on_sc=False),
    out_type=jax.ShapeDtypeStruct((B,), jnp.float32),
    scratch_types=[
        pltpu.VMEM((NCHUNK, CHUNK), jnp.int32),
        pltpu.VMEM((NCHUNK, CHUNK), jnp.int32),
        pltpu.VMEM((BPW, F), jnp.float32),
        pltpu.VMEM((BPW, F), jnp.float32),
        pltpu.VMEM((BPW,), jnp.float32),
        pltpu.VMEM((LANES * LANES,), jnp.float32),
        pltpu.SemaphoreType.DMA,
    ],
)
def _mf_kernel(user_hbm, item_hbm, uf_hbm, if_hbm, out_hbm,
               uidx, iidx, urows, vrows, outv, tpose, sem):
    wid = lax.axis_index("s") * NC + lax.axis_index("c")
    base = wid * BPW

    for j in range(NCHUNK):
        pltpu.sync_copy(user_hbm.at[pl.ds(base + j * CHUNK, CHUNK)], uidx.at[j])
        pltpu.sync_copy(item_hbm.at[pl.ds(base + j * CHUNK, CHUNK)], iidx.at[j])

    copies = []
    for j in range(NCHUNK):
        copies.append(pltpu.async_copy(
            uf_hbm.at[uidx.at[j]], urows.at[pl.ds(j * CHUNK, CHUNK)], sem))
        copies.append(pltpu.async_copy(
            if_hbm.at[iidx.at[j]], vrows.at[pl.ds(j * CHUNK, CHUNK)], sem))
    for cp in copies:
        cp.wait()

    lane = lax.iota(jnp.int32, LANES)

    def xlane(x, idx):
        return lax.gather(
            x, idx[:, None],
            lax.GatherDimensionNumbers(
                offset_dims=(), collapsed_slice_dims=(0,),
                start_index_map=(0,)),
            slice_sizes=(1,),
            mode=lax.GatherScatterMode.PROMISE_IN_BOUNDS)

    perms = [lane ^ d for d in (8, 4, 2, 1)]

    def block(bi, carry):
        rbase = bi * LANES
        acc = jnp.zeros((LANES,), jnp.float32)
        for r in range(LANES):
            row = rbase + r
            u0 = urows[row, pl.ds(0, LANES)]
            u1 = urows[row, pl.ds(LANES, LANES)]
            v0 = vrows[row, pl.ds(0, LANES)]
            v1 = vrows[row, pl.ds(LANES, LANES)]
            p = u0 * v0 + u1 * v1
            for pm in perms:
                p = p + xlane(p, pm)
            acc = jnp.where(lane == r, p, acc)
        outv[pl.ds(rbase, LANES)] = acc
        return carry

    lax.fori_loop(0, NBLK, block, 0)

    pltpu.sync_copy(outv, out_hbm.at[pl.ds(base, BPW)])


def kernel(user, item, user_factors, item_factors):
    return _mf_kernel(user.astype(jnp.int32), item.astype(jnp.int32),
                      user_factors, item_factors)
